# baseline (speedup 1.0000x reference)
import jax
import jax.numpy as jnp
from jax.experimental import pallas as pl
from jax.experimental.pallas import tpu as pltpu
from jax.experimental.pallas import tpu_sc as plsc

_NUM_CHUNKS = 5
_BLOCK_N = 200
_WINDOW = 128


def _sc_gather(feat, idx_flat, window):
    m = idx_flat.shape[1]
    c = feat.shape[1]
    mesh = plsc.VectorSubcoreMesh(core_axis_name="core", subcore_axis_name="subcore")

    @pl.kernel(out_type=jax.ShapeDtypeStruct((m, c), feat.dtype), mesh=mesh)
    def gather_kernel(x_hbm, i_hbm, o_hbm):
        def body(i_vmem, o_vmem):
            pltpu.sync_copy(x_hbm.at[i_vmem.at[0]], o_vmem)

        pltpu.emit_pipeline(
            body,
            grid=(m // window,),
            in_specs=[pl.BlockSpec((1, window), lambda i: (0, i))],
            out_specs=[pl.BlockSpec((window, c), lambda i: (i, 0))],
            core_axis_name=("core", "subcore"),
            dimension_semantics=(pltpu.PARALLEL,),
        )(i_hbm, o_hbm)

    return gather_kernel(feat, idx_flat)


def _tc_matmul_chunk(gathered, w, a, row0, prev, n_total, block_n):
    chunk, k, c_in = gathered.shape
    c_mid = w.shape[2]
    c_add = a.shape[2]
    c_out = (c_in + c_add) * c_mid
    blk0 = row0 // block_n

    def body(g_ref, w_ref, a_ref, *rest):
        o_ref = rest[-1]
        og = jax.lax.dot_general(
            g_ref[...], w_ref[...], (((1,), (1,)), ((0,), (0,))),
            preferred_element_type=jnp.float32,
        )
        oa = jax.lax.dot_general(
            a_ref[...], w_ref[...], (((1,), (1,)), ((0,), (0,))),
            preferred_element_type=jnp.float32,
        )
        o_ref[:, : c_in * c_mid] = og.reshape(block_n, c_in * c_mid)
        o_ref[:, c_in * c_mid :] = oa.reshape(block_n, c_add * c_mid)

    in_specs = [
        pl.BlockSpec((block_n, k, c_in), lambda i: (i, 0, 0)),
        pl.BlockSpec((block_n, k, c_mid), lambda i: (i, 0, 0)),
        pl.BlockSpec((block_n, k, c_add), lambda i: (i, 0, 0)),
    ]
    operands = [gathered, w, a]
    aliases = {}
    if prev is not None:
        in_specs.append(pl.BlockSpec(memory_space=pl.ANY))
        operands.append(prev)
        aliases = {3: 0}

    return pl.pallas_call(
        body,
        grid=(chunk // block_n,),
        in_specs=in_specs,
        out_specs=pl.BlockSpec((block_n, c_out), lambda i, b=blk0: (i + b, 0)),
        out_shape=jax.ShapeDtypeStruct((n_total, c_out), jnp.float32),
        input_output_aliases=aliases,
    )(*operands)


def kernel(input_features, neighbor_inds, weightnet, additional_features):
    b, n, c_in = input_features.shape
    k = neighbor_inds.shape[2]
    c_mid = weightnet.shape[3]
    c_add = additional_features.shape[3]

    feat = input_features.reshape(n, c_in)
    idx_flat = neighbor_inds.reshape(1, n * k)
    w3 = weightnet.reshape(n, k, c_mid)
    a3 = additional_features.reshape(n, k, c_add)

    chunk = n // _NUM_CHUNKS
    gathers = [
        _sc_gather(
            feat,
            jax.lax.slice(idx_flat, (0, j * chunk * k), (1, (j + 1) * chunk * k)),
            window=_WINDOW,
        ).reshape(chunk, k, c_in)
        for j in range(_NUM_CHUNKS)
    ]
    out = None
    for j in range(_NUM_CHUNKS):
        row0 = j * chunk
        out = _tc_matmul_chunk(
            gathers[j],
            jax.lax.slice(w3, (row0, 0, 0), (row0 + chunk, k, c_mid)),
            jax.lax.slice(a3, (row0, 0, 0), (row0 + chunk, k, c_add)),
            row0,
            out,
            n,
            _BLOCK_N,
        )
    return out.reshape(b, n, (c_in + c_add) * c_mid)
